# baseline (speedup 1.0000x reference)
import jax
import jax.numpy as jnp
from jax import lax
from jax.experimental import pallas as pl
from jax.experimental.pallas import tpu as pltpu
from jax.experimental.pallas import tpu_sc as plsc

_B, _C, _H, _W = 16, 768, 8, 256
_HW = _H * _W
_NCIR = 4
_L = 16
_NW = 32
_CPW = _C // _NW


def _sc_body(x_hbm, circ_hbm, out_hbm, xbuf_v, circ_v, sem):
    wid = lax.axis_index("s") * 2 + lax.axis_index("c")
    c0 = wid * _CPW
    for k in range(_NCIR):
        pltpu.sync_copy(circ_hbm.at[pl.ds((k * _C + c0) * _L, _CPW * _L)],
                        circ_v.at[pl.ds(k * _CPW * _L, _CPW * _L)])

    def per_channel(ci, carry):
        c = c0 + ci
        copies = [pltpu.async_copy(x_hbm.at[b, c], xbuf_v.at[b], sem)
                  for b in range(_B)]
        s = [circ_v[pl.ds((k * _CPW + ci) * _L, _L)] for k in range(_NCIR)]
        for cp in copies:
            cp.wait()

        def add(j, carry2):
            p = j * _L + lax.iota(jnp.int32, _L)
            w = lax.bitwise_and(p, _W - 1)
            h = lax.shift_right_logical(p, 8)
            d = jnp.minimum(jnp.minimum(h, (_H - 1) - h),
                            jnp.minimum(w, (_W - 1) - w))
            a = jnp.where(d == 0, s[0],
                          jnp.where(d == 1, s[1],
                                    jnp.where(d == 2, s[2], s[3])))
            for b in range(_B):
                xbuf_v[b, pl.ds(j * _L, _L)] = xbuf_v[b, pl.ds(j * _L, _L)] + a
            return carry2
        lax.fori_loop(0, _HW // _L, add, 0)

        outs = [pltpu.async_copy(xbuf_v.at[b], out_hbm.at[b, c], sem)
                for b in range(_B)]
        for cp in outs:
            cp.wait()
        return carry
    lax.fori_loop(0, _CPW, per_channel, 0)


def kernel(x, circles):
    xf = x.astype(jnp.float32).reshape(_B, _C, _HW)
    circ_rep = jnp.broadcast_to(
        circles.astype(jnp.float32).reshape(_NCIR * _C, 1), (_NCIR * _C, _L)
    ).reshape(_NCIR * _C * _L)
    mesh = plsc.VectorSubcoreMesh(core_axis_name="c", subcore_axis_name="s")
    run = pl.kernel(
        _sc_body,
        mesh=mesh,
        out_type=jax.ShapeDtypeStruct((_B, _C, _HW), jnp.float32),
        scratch_types=[
            pltpu.VMEM((_B, _HW), jnp.float32),
            pltpu.VMEM((_NCIR * _CPW * _L,), jnp.float32),
            pltpu.SemaphoreType.DMA,
        ],
    )
    out = run(xf, circ_rep)
    return out.reshape(_B, _C, _H, _W)


# speedup vs baseline: 1.2568x; 1.2568x over previous
import jax
import jax.numpy as jnp
from jax import lax
from jax.experimental import pallas as pl
from jax.experimental.pallas import tpu as pltpu
from jax.experimental.pallas import tpu_sc as plsc

_B, _C, _H, _W = 16, 768, 8, 256
_HW = _H * _W
_NCIR = 4
_L = 16
_NW = 32
_CPW = _C // _NW
_NSLOT = 3


def _sc_body(x_hbm, circ_hbm, out_hbm, xb0, xb1, xb2, circ_v,
             sem_in, sem_o0, sem_o1, sem_o2):
    wid = lax.axis_index("s") * 2 + lax.axis_index("c")
    c0 = wid * _CPW
    bufs = [xb0, xb1, xb2]
    osems = [sem_o0, sem_o1, sem_o2]
    for k in range(_NCIR):
        pltpu.sync_copy(circ_hbm.at[pl.ds((k * _C + c0) * _L, _CPW * _L)],
                        circ_v.at[pl.ds(k * _CPW * _L, _CPW * _L)])

    def fire_in(ci):
        return pltpu.async_copy(x_hbm.at[:, c0 + ci],
                                bufs[ci % _NSLOT], sem_in)

    def fire_out(ci):
        return pltpu.async_copy(bufs[ci % _NSLOT],
                                out_hbm.at[:, c0 + ci], osems[ci % _NSLOT])

    def comp(ci):
        buf = bufs[ci % _NSLOT]
        s = [circ_v[pl.ds((k * _CPW + ci) * _L, _L)] for k in range(_NCIR)]

        def add(j, carry):
            p = j * _L + lax.iota(jnp.int32, _L)
            w = lax.bitwise_and(p, _W - 1)
            h = lax.shift_right_logical(p, 8)
            d = jnp.minimum(jnp.minimum(h, (_H - 1) - h),
                            jnp.minimum(w, (_W - 1) - w))
            a = jnp.where(d == 0, s[0],
                          jnp.where(d == 1, s[1],
                                    jnp.where(d == 2, s[2], s[3])))
            for b in range(_B):
                plsc.addupdate(buf.at[b, pl.ds(j * _L, _L)], a)
            return carry
        lax.fori_loop(0, _HW // _L, add, 0)

    in_cp = [None] * _CPW
    out_cp = [None] * _CPW
    in_cp[0] = fire_in(0)
    in_cp[1] = fire_in(1)
    for ci in range(_CPW):
        in_cp[ci].wait()
        comp(ci)
        out_cp[ci] = fire_out(ci)
        if ci + 2 < _CPW:
            if ci >= 1:
                out_cp[ci - 1].wait()
            in_cp[ci + 2] = fire_in(ci + 2)
    out_cp[_CPW - 3].wait()
    out_cp[_CPW - 2].wait()
    out_cp[_CPW - 1].wait()


def kernel(x, circles):
    xf = x.astype(jnp.float32).reshape(_B, _C, _HW)
    circ_rep = jnp.broadcast_to(
        circles.astype(jnp.float32).reshape(_NCIR * _C, 1), (_NCIR * _C, _L)
    ).reshape(_NCIR * _C * _L)
    mesh = plsc.VectorSubcoreMesh(core_axis_name="c", subcore_axis_name="s")
    run = pl.kernel(
        _sc_body,
        mesh=mesh,
        out_type=jax.ShapeDtypeStruct((_B, _C, _HW), jnp.float32),
        scratch_types=[
            pltpu.VMEM((_B, _HW), jnp.float32),
            pltpu.VMEM((_B, _HW), jnp.float32),
            pltpu.VMEM((_B, _HW), jnp.float32),
            pltpu.VMEM((_NCIR * _CPW * _L,), jnp.float32),
            pltpu.SemaphoreType.DMA,
            pltpu.SemaphoreType.DMA,
            pltpu.SemaphoreType.DMA,
            pltpu.SemaphoreType.DMA,
        ],
    )
    out = run(xf, circ_rep)
    return out.reshape(_B, _C, _H, _W)


# speedup vs baseline: 3.6136x; 2.8753x over previous
import jax
import jax.numpy as jnp
from jax import lax
from jax.experimental import pallas as pl
from jax.experimental.pallas import tpu as pltpu
from jax.experimental.pallas import tpu_sc as plsc

_B, _C, _H, _W = 16, 768, 8, 256
_HW = _H * _W
_NCIR = 4
_L = 16
_NW = 32
_CPW = _C // _NW
_NSLOT = 3
_WL = _W // _L


def _sc_body(x_hbm, circ_hbm, out_hbm, xb0, xb1, xb2, circ_v,
             sem_in, sem_o0, sem_o1, sem_o2):
    wid = lax.axis_index("s") * 2 + lax.axis_index("c")
    c0 = wid * _CPW
    bufs = [xb0, xb1, xb2]
    osems = [sem_o0, sem_o1, sem_o2]
    for k in range(_NCIR):
        pltpu.sync_copy(circ_hbm.at[pl.ds((k * _C + c0) * _L, _CPW * _L)],
                        circ_v.at[pl.ds(k * _CPW * _L, _CPW * _L)])

    def fire_in(ci):
        return pltpu.async_copy(x_hbm.at[:, c0 + ci],
                                bufs[ci % _NSLOT], sem_in)

    def fire_out(ci):
        return pltpu.async_copy(bufs[ci % _NSLOT],
                                out_hbm.at[:, c0 + ci], osems[ci % _NSLOT])

    def comp(ci):
        buf = bufs[ci % _NSLOT]
        s = [circ_v[pl.ds((k * _CPW + ci) * _L, _L)] for k in range(_NCIR)]

        def add(j, carry):
            h = lax.shift_right_logical(j, 4)
            jw = lax.bitwise_and(j, _WL - 1)
            w = jw * _L + lax.iota(jnp.int32, _L)
            d = jnp.minimum(jnp.minimum(h, (_H - 1) - h),
                            jnp.minimum(w, (_W - 1) - w))
            a = jnp.where(d == 0, s[0],
                          jnp.where(d == 1, s[1],
                                    jnp.where(d == 2, s[2], s[3])))
            for b in range(_B):
                plsc.addupdate(buf.at[b, h, pl.ds(jw * _L, _L)], a)
            return carry
        lax.fori_loop(0, _HW // _L, add, 0)

    in_cp = [None] * _CPW
    out_cp = [None] * _CPW
    in_cp[0] = fire_in(0)
    in_cp[1] = fire_in(1)
    for ci in range(_CPW):
        in_cp[ci].wait()
        comp(ci)
        out_cp[ci] = fire_out(ci)
        if ci + 2 < _CPW:
            if ci >= 1:
                out_cp[ci - 1].wait()
            in_cp[ci + 2] = fire_in(ci + 2)
    out_cp[_CPW - 3].wait()
    out_cp[_CPW - 2].wait()
    out_cp[_CPW - 1].wait()


def kernel(x, circles):
    circ_rep = jnp.broadcast_to(
        circles.astype(jnp.float32).reshape(_NCIR * _C, 1), (_NCIR * _C, _L)
    ).reshape(_NCIR * _C * _L)
    mesh = plsc.VectorSubcoreMesh(core_axis_name="c", subcore_axis_name="s")
    run = pl.kernel(
        _sc_body,
        mesh=mesh,
        out_type=jax.ShapeDtypeStruct((_B, _C, _H, _W), jnp.float32),
        scratch_types=[
            pltpu.VMEM((_B, _H, _W), jnp.float32),
            pltpu.VMEM((_B, _H, _W), jnp.float32),
            pltpu.VMEM((_B, _H, _W), jnp.float32),
            pltpu.VMEM((_NCIR * _CPW * _L,), jnp.float32),
            pltpu.SemaphoreType.DMA,
            pltpu.SemaphoreType.DMA,
            pltpu.SemaphoreType.DMA,
            pltpu.SemaphoreType.DMA,
        ],
    )
    return run(x.astype(jnp.float32), circ_rep)
